# baseline (speedup 1.0000x reference)
import functools
import jax
import jax.numpy as jnp
from jax import lax
from jax.experimental import pallas as pl
from jax.experimental.pallas import tpu as pltpu
from jax.experimental.pallas import tpu_sc as plsc


def kernel(membership, table):
    B, S, D = membership.shape
    E = table.shape[1]
    N = B * S * D
    NW = 32
    NJ = N // 4
    PERW = NJ // NW
    G = 128

    m4 = membership.reshape(NJ, 4).astype(jnp.int32)
    j1 = m4[:, 0] + 2 * m4[:, 1] + 4 * m4[:, 2] + 8 * m4[:, 3]

    bits = jnp.arange(16, dtype=jnp.int32)
    combo = jnp.stack([(bits >> k) & 1 for k in range(4)], axis=1)
    lut = table[combo].reshape(16, 4 * E)

    mesh = plsc.VectorSubcoreMesh(core_axis_name="c", subcore_axis_name="s")

    @functools.partial(
        pl.kernel,
        mesh=mesh,
        out_type=jax.ShapeDtypeStruct((NJ, 4 * E), jnp.float32),
        scratch_types=[
            pltpu.VMEM((G,), jnp.int32),
            pltpu.VMEM((G, 4 * E), jnp.float32),
            pltpu.SemaphoreType.DMA,
        ],
    )
    def sc_lookup(j_hbm, lut_hbm, out_hbm, j_v, rows_v, sem):
        wid = lax.axis_index("s") * 2 + lax.axis_index("c")
        base = wid * PERW

        def step(i, carry):
            off = pl.multiple_of(base + i * G, G)
            pltpu.sync_copy(j_hbm.at[pl.ds(off, G)], j_v)
            pltpu.async_copy(lut_hbm.at[j_v], rows_v, sem).wait()
            pltpu.sync_copy(rows_v, out_hbm.at[pl.ds(off, G)])
            return carry

        lax.fori_loop(0, PERW // G, step, 0)

    out2 = sc_lookup(j1, lut)
    return out2.reshape(B, S, D, E)


# speedup vs baseline: 1.0016x; 1.0016x over previous
import functools
import jax
import jax.numpy as jnp
from jax import lax
from jax.experimental import pallas as pl
from jax.experimental.pallas import tpu as pltpu
from jax.experimental.pallas import tpu_sc as plsc


def kernel(membership, table):
    B, S, D = membership.shape
    E = table.shape[1]
    N = B * S * D
    NW = 32
    NJ = N // 4
    PERW = NJ // NW
    CH = 256
    NCH = PERW // CH

    m4 = membership.reshape(NJ, 4).astype(jnp.int32)
    j1 = m4[:, 0] + 2 * m4[:, 1] + 4 * m4[:, 2] + 8 * m4[:, 3]

    bits = jnp.arange(16, dtype=jnp.int32)
    combo = jnp.stack([(bits >> k) & 1 for k in range(4)], axis=1)
    lut = table[combo].reshape(16, 4 * E)

    mesh = plsc.VectorSubcoreMesh(core_axis_name="c", subcore_axis_name="s")

    @functools.partial(
        pl.kernel,
        mesh=mesh,
        out_type=jax.ShapeDtypeStruct((NJ, 4 * E), jnp.float32),
        scratch_types=[
            pltpu.VMEM((CH,), jnp.int32),
            pltpu.VMEM((CH,), jnp.int32),
            pltpu.VMEM((CH, 4 * E), jnp.float32),
            pltpu.VMEM((CH, 4 * E), jnp.float32),
            pltpu.SemaphoreType.DMA,
        ],
    )
    def sc_lookup(j_hbm, lut_hbm, out_hbm, j0, j1, rows0, rows1, sem):
        wid = lax.axis_index("s") * 2 + lax.axis_index("c")
        base = wid * PERW

        def gather(j_v, rows_v):
            h0 = pltpu.async_copy(
                lut_hbm.at[j_v.at[pl.ds(0, 128)]], rows_v.at[pl.ds(0, 128)], sem)
            h1 = pltpu.async_copy(
                lut_hbm.at[j_v.at[pl.ds(128, 128)]], rows_v.at[pl.ds(128, 128)], sem)
            return h0, h1

        def load_j(chunk, j_v):
            pltpu.sync_copy(j_hbm.at[pl.ds(pl.multiple_of(base + chunk * CH, CH), CH)], j_v)

        def store(chunk, rows_v):
            pltpu.sync_copy(rows_v, out_hbm.at[pl.ds(pl.multiple_of(base + chunk * CH, CH), CH)])

        load_j(0, j0)

        def step(k, carry):
            a = 2 * k
            ha = gather(j0, rows0)
            @pl.when(k > 0)
            def _():
                store(a - 1, rows1)
            load_j(a + 1, j1)
            ha[0].wait()
            ha[1].wait()
            hc = gather(j1, rows1)
            store(a, rows0)
            load_j(a + 2, j0)
            hc[0].wait()
            hc[1].wait()
            return carry

        lax.fori_loop(0, (NCH - 1) // 2, step, 0)
        store(NCH - 2, rows1)
        ht = gather(j0, rows0)
        ht[0].wait()
        ht[1].wait()
        store(NCH - 1, rows0)

    out2 = sc_lookup(j1, lut)
    return out2.reshape(B, S, D, E)
